# baseline (speedup 1.0000x reference)
import functools

import jax
import jax.numpy as jnp
from jax import lax
from jax.experimental import pallas as pl
from jax.experimental.pallas import tpu as pltpu

R = 8


def _ce_kernel(idx_ref, tgt_ref, *refs):
    row_refs = refs[:R]
    out_ref, loss_ref, accum_ref = refs[R], refs[R + 1], refs[R + 2]
    i = pl.program_id(0)
    nb = pl.num_programs(0)

    @pl.when(i == 0)
    def _():
        accum_ref[0] = 0.0

    c = row_refs[0].shape[2]
    rows = jnp.concatenate([r[...].reshape(1, c) for r in row_refs], axis=0)
    out_ref[...] = rows

    m = jnp.max(rows, axis=1, keepdims=True)
    s = jnp.sum(jnp.exp(rows - m), axis=1)
    logz = m[:, 0] + jnp.log(s)

    col = lax.broadcasted_iota(jnp.int32, (R, c), 1)
    tvec = jnp.stack([tgt_ref[i * R + r] for r in range(R)])
    picked = jnp.sum(jnp.where(col == tvec[:, None], rows, 0.0), axis=1)

    accum_ref[0] += jnp.sum(logz - picked)

    @pl.when(i == nb - 1)
    def _():
        loss_ref[0] = accum_ref[0] / (nb * R)


@jax.jit
def kernel(input_tensor, target_tensor, table):
    b, t = input_tensor.shape
    n = b * t
    v, c = table.shape
    idx = input_tensor.reshape(n)
    tgt = target_tensor.reshape(n)
    nb = n // R

    table3 = table.reshape(v, 1, c)

    def row_spec(r):
        return pl.BlockSpec(
            (1, 1, c), lambda i, idx_ref, tgt_ref, r=r: (idx_ref[i * R + r], 0, 0)
        )

    grid_spec = pltpu.PrefetchScalarGridSpec(
        num_scalar_prefetch=2,
        grid=(nb,),
        in_specs=[row_spec(r) for r in range(R)],
        out_specs=[
            pl.BlockSpec((R, c), lambda i, idx_ref, tgt_ref: (i, 0)),
            pl.BlockSpec(memory_space=pltpu.SMEM),
        ],
        scratch_shapes=[pltpu.SMEM((1,), jnp.float32)],
    )

    logits, loss = pl.pallas_call(
        _ce_kernel,
        grid_spec=grid_spec,
        out_shape=[
            jax.ShapeDtypeStruct((n, c), jnp.float32),
            jax.ShapeDtypeStruct((1,), jnp.float32),
        ],
    )(idx, tgt, *([table3] * R))
    return logits, loss[0]


# speedup vs baseline: 1.3831x; 1.3831x over previous
import functools

import jax
import jax.numpy as jnp
from jax import lax
from jax.experimental import pallas as pl
from jax.experimental.pallas import tpu as pltpu

R = 16


def _ce_kernel(idx_ref, tgt_ref, *refs):
    row_refs = refs[:R]
    out_ref, loss_ref, accum_ref = refs[R], refs[R + 1], refs[R + 2]
    i = pl.program_id(0)
    nb = pl.num_programs(0)

    @pl.when(i == 0)
    def _():
        accum_ref[0] = 0.0

    c = row_refs[0].shape[2]
    rows = jnp.concatenate([r[...].reshape(1, c) for r in row_refs], axis=0)
    out_ref[...] = rows

    m = jnp.max(rows, axis=1, keepdims=True)
    s = jnp.sum(jnp.exp(rows - m), axis=1)
    logz = m[:, 0] + jnp.log(s)

    col = lax.broadcasted_iota(jnp.int32, (R, c), 1)
    tvec = jnp.stack([tgt_ref[i * R + r] for r in range(R)])
    picked = jnp.sum(jnp.where(col == tvec[:, None], rows, 0.0), axis=1)

    accum_ref[0] += jnp.sum(logz - picked)

    @pl.when(i == nb - 1)
    def _():
        loss_ref[0] = accum_ref[0] / (nb * R)


@jax.jit
def kernel(input_tensor, target_tensor, table):
    b, t = input_tensor.shape
    n = b * t
    v, c = table.shape
    idx = input_tensor.reshape(n)
    tgt = target_tensor.reshape(n)
    nb = n // R

    table3 = table.reshape(v, 1, c)

    def row_spec(r):
        return pl.BlockSpec(
            (1, 1, c), lambda i, idx_ref, tgt_ref, r=r: (idx_ref[i * R + r], 0, 0)
        )

    grid_spec = pltpu.PrefetchScalarGridSpec(
        num_scalar_prefetch=2,
        grid=(nb,),
        in_specs=[row_spec(r) for r in range(R)],
        out_specs=[
            pl.BlockSpec((R, c), lambda i, idx_ref, tgt_ref: (i, 0)),
            pl.BlockSpec(memory_space=pltpu.SMEM),
        ],
        scratch_shapes=[pltpu.SMEM((1,), jnp.float32)],
    )

    logits, loss = pl.pallas_call(
        _ce_kernel,
        grid_spec=grid_spec,
        out_shape=[
            jax.ShapeDtypeStruct((n, c), jnp.float32),
            jax.ShapeDtypeStruct((1,), jnp.float32),
        ],
    )(idx, tgt, *([table3] * R))
    return logits, loss[0]


# speedup vs baseline: 1.6811x; 1.2155x over previous
import functools

import jax
import jax.numpy as jnp
from jax import lax
from jax.experimental import pallas as pl
from jax.experimental.pallas import tpu as pltpu

R = 32


def _ce_kernel(idx_ref, tgt_ref, *refs):
    row_refs = refs[:R]
    out_ref, loss_ref, accum_ref = refs[R], refs[R + 1], refs[R + 2]
    i = pl.program_id(0)
    nb = pl.num_programs(0)

    @pl.when(i == 0)
    def _():
        accum_ref[0] = 0.0

    c = row_refs[0].shape[2]
    rows = jnp.concatenate([r[...].reshape(1, c) for r in row_refs], axis=0)
    out_ref[...] = rows

    m = jnp.max(rows, axis=1, keepdims=True)
    s = jnp.sum(jnp.exp(rows - m), axis=1)
    logz = m[:, 0] + jnp.log(s)

    col = lax.broadcasted_iota(jnp.int32, (R, c), 1)
    tvec = jnp.stack([tgt_ref[i * R + r] for r in range(R)])
    picked = jnp.sum(jnp.where(col == tvec[:, None], rows, 0.0), axis=1)

    accum_ref[0] += jnp.sum(logz - picked)

    @pl.when(i == nb - 1)
    def _():
        loss_ref[0] = accum_ref[0] / (nb * R)


@jax.jit
def kernel(input_tensor, target_tensor, table):
    b, t = input_tensor.shape
    n = b * t
    v, c = table.shape
    idx = input_tensor.reshape(n)
    tgt = target_tensor.reshape(n)
    nb = n // R

    table3 = table.reshape(v, 1, c)

    def row_spec(r):
        return pl.BlockSpec(
            (1, 1, c), lambda i, idx_ref, tgt_ref, r=r: (idx_ref[i * R + r], 0, 0)
        )

    grid_spec = pltpu.PrefetchScalarGridSpec(
        num_scalar_prefetch=2,
        grid=(nb,),
        in_specs=[row_spec(r) for r in range(R)],
        out_specs=[
            pl.BlockSpec((R, c), lambda i, idx_ref, tgt_ref: (i, 0)),
            pl.BlockSpec(memory_space=pltpu.SMEM),
        ],
        scratch_shapes=[pltpu.SMEM((1,), jnp.float32)],
    )

    logits, loss = pl.pallas_call(
        _ce_kernel,
        grid_spec=grid_spec,
        out_shape=[
            jax.ShapeDtypeStruct((n, c), jnp.float32),
            jax.ShapeDtypeStruct((1,), jnp.float32),
        ],
    )(idx, tgt, *([table3] * R))
    return logits, loss[0]
